# baseline (speedup 1.0000x reference)
import functools

import jax
import jax.numpy as jnp
from jax import lax
from jax.experimental import pallas as pl
from jax.experimental.pallas import tpu as pltpu
from jax.experimental.pallas import tpu_sc as plsc

N = 10000
E = 320000
D = 128
G = 128
NC, NS = 2, 16
NW = NC * NS
CH = 128
K = 80
EPW = K * CH
EPAD = NW * EPW
NPAD = 10240
RPT = NPAD // (NS * CH)
RB = 2000
NBLK = N // RB


def _sc_degree(dstp):
    mesh = plsc.VectorSubcoreMesh(core_axis_name="c", subcore_axis_name="s")

    @functools.partial(
        pl.kernel,
        out_type=jax.ShapeDtypeStruct((NC, NPAD), jnp.float32),
        mesh=mesh,
        scratch_types=[
            pltpu.VMEM((K, CH), jnp.int32),
            pltpu.VMEM((CH,), jnp.float32),
            pltpu.VMEM((NPAD // NS,), jnp.float32),
            pltpu.VMEM_SHARED((NPAD,), jnp.float32),
        ],
    )
    def k(dst_hbm, out_hbm, idx, ones, zb, acc):
        cid = lax.axis_index("c")
        sid = lax.axis_index("s")
        w = cid * NS + sid
        one16 = jnp.ones((16,), jnp.float32)
        zero16 = jnp.zeros((16,), jnp.float32)
        for l in range(CH // 16):
            ones[pl.ds(l * 16, 16)] = one16

        def zfill(i, c):
            zb[pl.ds(i * 16, 16)] = zero16
            return c
        lax.fori_loop(0, (NPAD // NS) // 16, zfill, 0)
        seg = NPAD // NS
        pltpu.sync_copy(zb, acc.at[pl.ds(sid * seg, seg)])
        pltpu.sync_copy(dst_hbm.at[w], idx)
        plsc.subcore_barrier()

        def body(j, c):
            pltpu.sync_copy(ones, acc.at[idx.at[j]], add=True)
            return c
        lax.fori_loop(0, K, body, 0)
        plsc.subcore_barrier()
        pltpu.sync_copy(acc.at[pl.ds(sid * seg, seg)],
                        out_hbm.at[cid, pl.ds(sid * seg, seg)])

    return k(dstp)


def _sc_scatter(u, edges):
    mesh = plsc.VectorSubcoreMesh(core_axis_name="c", subcore_axis_name="s")

    @functools.partial(
        pl.kernel,
        out_type=jax.ShapeDtypeStruct((NC, NPAD, D), jnp.float32),
        mesh=mesh,
        scratch_types=[
            pltpu.VMEM((4, 2, CH), jnp.int32),
            pltpu.VMEM((CH, D), jnp.float32),
            pltpu.VMEM((CH, D), jnp.float32),
            pltpu.VMEM_SHARED((NPAD, D), jnp.float32),
            pltpu.SemaphoreType.DMA,
            pltpu.SemaphoreType.DMA,
            pltpu.SemaphoreType.DMA,
            pltpu.SemaphoreType.DMA,
            pltpu.SemaphoreType.DMA,
            pltpu.SemaphoreType.DMA,
        ],
    )
    def k(u_hbm, e_hbm, out_hbm, ib, rows0, rows1, acc,
          sg0, sg1, si0, si1, si2, si3):
        cid = lax.axis_index("c")
        sid = lax.axis_index("s")
        w = cid * NS + sid
        zero16 = jnp.zeros((16,), jnp.float32)

        def zfill(i, c):
            rows0[i // (D // 16), pl.ds((i % (D // 16)) * 16, 16)] = zero16
            return c
        lax.fori_loop(0, CH * (D // 16), zfill, 0)
        for t in range(RPT):
            off = (sid * RPT + t) * CH
            pltpu.sync_copy(rows0, acc.at[pl.ds(off, CH)])
        plsc.subcore_barrier()

        gsem = (sg0, sg1)
        isem = (si0, si1, si2, si3)
        for j in range(4):
            pltpu.async_copy(e_hbm.at[w, j], ib.at[j], isem[j])
        for j in range(2):
            pltpu.make_async_copy(e_hbm.at[w, j], ib.at[j], isem[j]).wait()
            pltpu.async_copy(u_hbm.at[ib.at[j, 0]], (rows0, rows1)[j], gsem[j])

        def body(jj, c):
            for b in range(4):
                j = jj * 4 + b
                rows, gs = (rows0, rows1)[b % 2], gsem[b % 2]
                pltpu.make_async_copy(u_hbm.at[ib.at[b, 0]], rows, gs).wait()
                pltpu.sync_copy(rows, acc.at[ib.at[b, 1]], add=True)

                @pl.when(j + 2 < K)
                def _():
                    b2 = (b + 2) % 4
                    pltpu.make_async_copy(
                        e_hbm.at[w, j + 2], ib.at[b2], isem[b2]).wait()
                    pltpu.async_copy(u_hbm.at[ib.at[b2, 0]], rows, gs)

                @pl.when(j + 4 < K)
                def _():
                    pltpu.async_copy(e_hbm.at[w, j + 4], ib.at[b], isem[b])
            return c
        lax.fori_loop(0, K // 4, body, 0)
        plsc.subcore_barrier()
        for t in range(RPT):
            off = (sid * RPT + t) * CH
            pltpu.sync_copy(acc.at[pl.ds(off, CH)],
                            out_hbm.at[cid, pl.ds(off, CH)])

    return k(u, edges)


def _tc_prescale(x, W, degT):
    def body(x_ref, w_ref, deg_ref, u_ref):
        dg = deg_ref[:, 0:1] + deg_ref[:, 1:2] + 1.0
        dis = lax.rsqrt(dg)
        xw = jnp.dot(x_ref[...], w_ref[...],
                     preferred_element_type=jnp.float32,
                     precision=lax.Precision.HIGHEST)
        u_ref[...] = xw * dis

    return pl.pallas_call(
        body,
        grid=(NBLK,),
        in_specs=[
            pl.BlockSpec((RB, D), lambda i: (i, 0)),
            pl.BlockSpec((D, D), lambda i: (0, 0)),
            pl.BlockSpec((RB, NC), lambda i: (i, 0)),
        ],
        out_specs=pl.BlockSpec((RB, D), lambda i: (i, 0)),
        out_shape=jax.ShapeDtypeStruct((N, D), jnp.float32),
    )(x, W, degT)


def _tc_mid(accp, u1, degT, W2, b1r, ar, batch_r):
    def body(acc_ref, u_ref, deg_ref, w_ref, b_ref, a_ref, bt_ref, u2_ref, g_ref):
        i = pl.program_id(0)
        dg = deg_ref[:, 0:1] + deg_ref[:, 1:2] + 1.0
        dis = lax.rsqrt(dg)
        z = dis * (acc_ref[0] + acc_ref[1] + u_ref[...]) + b_ref[...]
        z = jnp.where(z >= 0, z, a_ref[...] * z)
        oh = (bt_ref[0] == lax.broadcasted_iota(jnp.int32, (G, RB), 0))
        gblk = jnp.dot(oh.astype(jnp.float32), z,
                       preferred_element_type=jnp.float32,
                       precision=lax.Precision.HIGHEST)

        @pl.when(i == 0)
        def _():
            g_ref[...] = gblk

        @pl.when(i > 0)
        def _():
            g_ref[...] = g_ref[...] + gblk

        u2_ref[...] = dis * jnp.dot(z, w_ref[...],
                                    preferred_element_type=jnp.float32,
                                    precision=lax.Precision.HIGHEST)

    return pl.pallas_call(
        body,
        grid=(NBLK,),
        in_specs=[
            pl.BlockSpec((NC, RB, D), lambda i: (0, i, 0)),
            pl.BlockSpec((RB, D), lambda i: (i, 0)),
            pl.BlockSpec((RB, NC), lambda i: (i, 0)),
            pl.BlockSpec((D, D), lambda i: (0, 0)),
            pl.BlockSpec((1, D), lambda i: (0, 0)),
            pl.BlockSpec((1, D), lambda i: (0, 0)),
            pl.BlockSpec((1, 1, RB), lambda i: (i, 0, 0)),
        ],
        out_specs=[
            pl.BlockSpec((RB, D), lambda i: (i, 0)),
            pl.BlockSpec((G, D), lambda i: (0, 0)),
        ],
        out_shape=[
            jax.ShapeDtypeStruct((N, D), jnp.float32),
            jax.ShapeDtypeStruct((G, D), jnp.float32),
        ],
    )(accp, u1, degT, W2, b1r, ar, batch_r)


def _tc_final(accp, u2, degT, b2r, ar, batch_r):
    def body(acc_ref, u_ref, deg_ref, b_ref, a_ref, bt_ref, z_ref, g_ref):
        i = pl.program_id(0)
        dg = deg_ref[:, 0:1] + deg_ref[:, 1:2] + 1.0
        dis = lax.rsqrt(dg)
        z = dis * (acc_ref[0] + acc_ref[1] + u_ref[...]) + b_ref[...]
        z = jnp.where(z >= 0, z, a_ref[...] * z)
        z_ref[...] = z
        oh = (bt_ref[0] == lax.broadcasted_iota(jnp.int32, (G, RB), 0))
        gblk = jnp.dot(oh.astype(jnp.float32), z,
                       preferred_element_type=jnp.float32,
                       precision=lax.Precision.HIGHEST)

        @pl.when(i == 0)
        def _():
            g_ref[...] = gblk

        @pl.when(i > 0)
        def _():
            g_ref[...] = g_ref[...] + gblk

    return pl.pallas_call(
        body,
        grid=(NBLK,),
        in_specs=[
            pl.BlockSpec((NC, RB, D), lambda i: (0, i, 0)),
            pl.BlockSpec((RB, D), lambda i: (i, 0)),
            pl.BlockSpec((RB, NC), lambda i: (i, 0)),
            pl.BlockSpec((1, D), lambda i: (0, 0)),
            pl.BlockSpec((1, D), lambda i: (0, 0)),
            pl.BlockSpec((1, 1, RB), lambda i: (i, 0, 0)),
        ],
        out_specs=[
            pl.BlockSpec((RB, D), lambda i: (i, 0)),
            pl.BlockSpec((G, D), lambda i: (0, 0)),
        ],
        out_shape=[
            jax.ShapeDtypeStruct((N, D), jnp.float32),
            jax.ShapeDtypeStruct((G, D), jnp.float32),
        ],
    )(accp, u2, degT, b2r, ar, batch_r)


def kernel(batch, x, edge_index, W1, b1, W2, b2, a):
    src, dst = edge_index[0], edge_index[1]
    pad = EPAD - E
    srcp = jnp.concatenate([src, jnp.zeros((pad,), src.dtype)]).reshape(NW, K, CH)
    dstp = jnp.concatenate([dst, jnp.full((pad,), N, dst.dtype)]).reshape(NW, K, CH)
    edges = jnp.stack([srcp, dstp], axis=2)

    degp = _sc_degree(dstp)
    degT = degp[:, :N].T
    b1r = b1.reshape(1, D)
    b2r = b2.reshape(1, D)
    ar = a.reshape(1, D)
    batch_r = batch.reshape(NBLK, 1, RB)

    u1 = _tc_prescale(x, W1, degT)
    acc1 = _sc_scatter(u1, edges)
    u2, g1 = _tc_mid(acc1[:, :N], u1, degT, W2, b1r, ar, batch_r)
    acc2 = _sc_scatter(u2, edges)
    z2, g2 = _tc_final(acc2[:, :N], u2, degT, b2r, ar, batch_r)
    return (z2, jnp.concatenate([g1, g2], axis=1))
